# Initial kernel scaffold; baseline (speedup 1.0000x reference)
import jax
import jax.numpy as jnp
from jax.experimental import pallas as pl


def kernel(x_user, x_item, W_user, b_user, W_item, b_item, edge_user_item, edge_item_user):
    raise NotImplementedError("write your pallas kernel here")


# speedup vs baseline: 5.1942x; 5.1942x over previous
import functools

import jax
import jax.numpy as jnp
from jax import lax
from jax.experimental import pallas as pl
from jax.experimental.pallas import tpu as pltpu
from jax.experimental.pallas import tpu_sc as plsc

N_USER = 5000
N_ITEM = 5000
N = N_USER + N_ITEM
D = 128
E_PER = 160000
E = 2 * E_PER
NUM_LAYERS = 3
ALPHA = 1.0 / (NUM_LAYERS + 1)

NC = 2
NS = 16
NW = NC * NS
K = 128
EPT = 10240
EPAD = EPT * NW
NB = EPT // K
NPAD = 10240
RPT = NPAD // NS

_sc_mesh = plsc.VectorSubcoreMesh(core_axis_name="c", subcore_axis_name="s")


def _lin_body(x_ref, w_ref, b_ref, o_ref):
    o_ref[...] = (
        jnp.dot(x_ref[0], w_ref[0], preferred_element_type=jnp.float32)
        + b_ref[0]
    )


def _lin_call(X, W, b):
    return pl.pallas_call(
        _lin_body,
        grid=(2, 5),
        in_specs=[
            pl.BlockSpec((1, 1000, D), lambda t, j: (t, j, 0)),
            pl.BlockSpec((1, D, D), lambda t, j: (t, 0, 0)),
            pl.BlockSpec((1, 1, D), lambda t, j: (t, 0, 0)),
        ],
        out_specs=pl.BlockSpec((1000, D), lambda t, j: (t * 5 + j, 0)),
        out_shape=jax.ShapeDtypeStruct((N, D), jnp.float32),
    )(X, W, b)


@functools.partial(
    pl.kernel,
    out_type=jax.ShapeDtypeStruct((NC, NPAD, D), jnp.float32),
    mesh=_sc_mesh,
    scratch_types=[
        pltpu.VMEM((K,), jnp.int32),
        pltpu.VMEM((K, D), jnp.float32),
        pltpu.VMEM_SHARED((NPAD, D), jnp.float32),
    ],
)
def _deg_kernel(col_hbm, degp_hbm, cidx_v, buf_v, acc_sh):
    c = lax.axis_index("c")
    s = lax.axis_index("s")
    wid = s * NC + c

    def _zero(j, _):
        for k in range(D // 16):
            buf_v[j, pl.ds(16 * k, 16)] = jnp.zeros((16,), jnp.float32)
        return 0

    lax.fori_loop(0, K, _zero, 0)
    rbase = pl.multiple_of(s * RPT, 8)
    for i in range(RPT // K):
        pltpu.sync_copy(buf_v, acc_sh.at[pl.ds(rbase + i * K, K)])

    def _ones(j, _):
        for k in range(D // 16):
            buf_v[j, pl.ds(16 * k, 16)] = jnp.ones((16,), jnp.float32)
        return 0

    lax.fori_loop(0, K, _ones, 0)
    plsc.subcore_barrier()

    def _body(bi, _):
        base = pl.multiple_of(wid * EPT + bi * K, 8)
        pltpu.sync_copy(col_hbm.at[pl.ds(base, K)], cidx_v)
        pltpu.sync_copy(buf_v, acc_sh.at[cidx_v], add=True)
        return 0

    lax.fori_loop(0, NB, _body, 0)
    plsc.subcore_barrier()
    pltpu.sync_copy(acc_sh.at[pl.ds(rbase, RPT)], degp_hbm.at[c].at[pl.ds(rbase, RPT)])


@functools.partial(
    pl.kernel,
    out_type=jax.ShapeDtypeStruct((NC, NPAD, D), jnp.float32),
    mesh=_sc_mesh,
    scratch_types=[
        pltpu.VMEM((K,), jnp.int32),
        pltpu.VMEM((K,), jnp.int32),
        pltpu.VMEM((K, D), jnp.float32),
        pltpu.VMEM_SHARED((NPAD, D), jnp.float32),
        pltpu.SemaphoreType.DMA,
    ],
)
def _prop_kernel(y_hbm, row_hbm, col_hbm, p_hbm, ridx_v, cidx_v, rows_v, acc_sh, sem):
    c = lax.axis_index("c")
    s = lax.axis_index("s")
    wid = s * NC + c

    def _zero(j, _):
        for k in range(D // 16):
            rows_v[j, pl.ds(16 * k, 16)] = jnp.zeros((16,), jnp.float32)
        return 0

    lax.fori_loop(0, K, _zero, 0)
    rbase = pl.multiple_of(s * RPT, 8)
    for i in range(RPT // K):
        pltpu.sync_copy(rows_v, acc_sh.at[pl.ds(rbase + i * K, K)])
    plsc.subcore_barrier()

    def _body(bi, _):
        base = pl.multiple_of(wid * EPT + bi * K, 8)
        pltpu.sync_copy(row_hbm.at[pl.ds(base, K)], ridx_v)
        pltpu.sync_copy(col_hbm.at[pl.ds(base, K)], cidx_v)
        pltpu.async_copy(y_hbm.at[ridx_v], rows_v, sem).wait()
        pltpu.sync_copy(rows_v, acc_sh.at[cidx_v], add=True)
        return 0

    lax.fori_loop(0, NB, _body, 0)
    plsc.subcore_barrier()
    pltpu.sync_copy(acc_sh.at[pl.ds(rbase, RPT)], p_hbm.at[c].at[pl.ds(rbase, RPT)])


def _s16_body(dp_ref, s_ref):
    d = dp_ref[0][:, 0:16] + dp_ref[1][:, 0:16]
    s_ref[...] = jnp.where(d > 0, lax.rsqrt(jnp.where(d > 0, d, 1.0)), 0.0)


def _s16_call(degp):
    return pl.pallas_call(
        _s16_body,
        grid=(NPAD // 1024,),
        in_specs=[pl.BlockSpec((2, 1024, D), lambda i: (0, i, 0))],
        out_specs=pl.BlockSpec((1024, 16), lambda i: (i, 0)),
        out_shape=jax.ShapeDtypeStruct((NPAD, 16), jnp.float32),
    )(degp)


def _y_body(h_ref, s_ref, y_ref):
    y_ref[...] = h_ref[...] * s_ref[:, 0:1]


def _y_call(h, s16):
    return pl.pallas_call(
        _y_body,
        grid=(10,),
        in_specs=[
            pl.BlockSpec((1000, D), lambda i: (i, 0)),
            pl.BlockSpec((1000, 16), lambda i: (i, 0)),
        ],
        out_specs=pl.BlockSpec((1000, D), lambda i: (i, 0)),
        out_shape=jax.ShapeDtypeStruct((N, D), jnp.float32),
    )(h, s16)


def _combine_body(p_ref, s_ref, o_ref, onew_ref, y_ref):
    sc = s_ref[:, 0:1]
    x = sc * (p_ref[0] + p_ref[1])
    onew_ref[...] = o_ref[...] + x
    y_ref[...] = sc * x


def _combine_call(p, s16, out):
    return pl.pallas_call(
        _combine_body,
        grid=(10,),
        in_specs=[
            pl.BlockSpec((2, 1000, D), lambda i: (0, i, 0)),
            pl.BlockSpec((1000, 16), lambda i: (i, 0)),
            pl.BlockSpec((1000, D), lambda i: (i, 0)),
        ],
        out_specs=[
            pl.BlockSpec((1000, D), lambda i: (i, 0)),
            pl.BlockSpec((1000, D), lambda i: (i, 0)),
        ],
        out_shape=[
            jax.ShapeDtypeStruct((N, D), jnp.float32),
            jax.ShapeDtypeStruct((N, D), jnp.float32),
        ],
    )(p, s16, out)


def _final_body(p_ref, s_ref, o_ref, z_ref):
    sc = s_ref[:, 0:1]
    z_ref[...] = ALPHA * (o_ref[...] + sc * (p_ref[0] + p_ref[1]))


def _final_call(p, s16, out):
    return pl.pallas_call(
        _final_body,
        grid=(10,),
        in_specs=[
            pl.BlockSpec((2, 1000, D), lambda i: (0, i, 0)),
            pl.BlockSpec((1000, 16), lambda i: (i, 0)),
            pl.BlockSpec((1000, D), lambda i: (i, 0)),
        ],
        out_specs=pl.BlockSpec((1000, D), lambda i: (i, 0)),
        out_shape=jax.ShapeDtypeStruct((N, D), jnp.float32),
    )(p, s16, out)


@jax.jit
def _impl(x_user, x_item, W_user, b_user, W_item, b_item, eui, eiu):
    X = jnp.stack([x_user, x_item])
    W = jnp.stack([W_user, W_item])
    b = jnp.stack([b_user, b_item])[:, None, :]
    h = _lin_call(X, W, b)

    npad = EPAD - E
    row = jnp.concatenate(
        [eui[0], eiu[0] + N_USER, jnp.zeros((npad,), jnp.int32)]
    )
    col = jnp.concatenate(
        [eui[1] + N_USER, eiu[1], jnp.full((npad,), N, jnp.int32)]
    )

    degp = _deg_kernel(col)
    s16 = _s16_call(degp)
    y = _y_call(h, s16)

    out = h
    for layer in range(NUM_LAYERS):
        p = _prop_kernel(y, row, col)
        if layer < NUM_LAYERS - 1:
            out, y = _combine_call(p, s16, out)
        else:
            z = _final_call(p, s16, out)
    return z[:N_USER], z[N_USER:]


def kernel(x_user, x_item, W_user, b_user, W_item, b_item,
           edge_user_item, edge_item_user):
    return _impl(x_user, x_item, W_user, b_user, W_item, b_item,
                 edge_user_item, edge_item_user)


# speedup vs baseline: 5.6320x; 1.0843x over previous
import functools

import jax
import jax.numpy as jnp
from jax import lax
from jax.experimental import pallas as pl
from jax.experimental.pallas import tpu as pltpu
from jax.experimental.pallas import tpu_sc as plsc

N_USER = 5000
N_ITEM = 5000
N = N_USER + N_ITEM
D = 128
E_PER = 160000
E = 2 * E_PER
NUM_LAYERS = 3
ALPHA = 1.0 / (NUM_LAYERS + 1)

NC = 2
NS = 16
NW = NC * NS
K = 32
EPT = 10240
EPAD = EPT * NW
NB = EPT // K
NPAD = 10240
RPT = NPAD // NS

_sc_mesh = plsc.VectorSubcoreMesh(core_axis_name="c", subcore_axis_name="s")


def _lin_body(x_ref, w_ref, b_ref, o_ref):
    o_ref[...] = (
        jnp.dot(x_ref[0], w_ref[0], preferred_element_type=jnp.float32)
        + b_ref[0]
    )


def _lin_call(X, W, b):
    return pl.pallas_call(
        _lin_body,
        grid=(2, 5),
        in_specs=[
            pl.BlockSpec((1, 1000, D), lambda t, j: (t, j, 0)),
            pl.BlockSpec((1, D, D), lambda t, j: (t, 0, 0)),
            pl.BlockSpec((1, 1, D), lambda t, j: (t, 0, 0)),
        ],
        out_specs=pl.BlockSpec((1000, D), lambda t, j: (t * 5 + j, 0)),
        out_shape=jax.ShapeDtypeStruct((N, D), jnp.float32),
    )(X, W, b)


@functools.partial(
    pl.kernel,
    out_type=jax.ShapeDtypeStruct((NC, NPAD, D), jnp.float32),
    mesh=_sc_mesh,
    scratch_types=[
        pltpu.VMEM((4, 2, K), jnp.int32),
        pltpu.VMEM((K, D), jnp.float32),
        pltpu.VMEM_SHARED((NPAD, D), jnp.float32),
        pltpu.SemaphoreType.DMA,
        pltpu.SemaphoreType.DMA,
        pltpu.SemaphoreType.DMA,
        pltpu.SemaphoreType.DMA,
        pltpu.SemaphoreType.DMA,
        pltpu.SemaphoreType.DMA,
    ],
)
def _deg_kernel(rc_hbm, degp_hbm, ir_v, buf_v, acc_sh,
                si0, si1, si2, si3, ss0, ss1):
    c = lax.axis_index("c")
    s = lax.axis_index("s")
    wid = s * NC + c
    gb0 = wid * NB
    si = (si0, si1, si2, si3)
    ss = (ss0, ss1)

    def _zero(j, _):
        for k in range(D // 16):
            buf_v[j, pl.ds(16 * k, 16)] = jnp.zeros((16,), jnp.float32)
        return 0

    lax.fori_loop(0, K, _zero, 0)
    rbase = pl.multiple_of(s * RPT, 8)
    for i in range(RPT // K):
        pltpu.sync_copy(buf_v, acc_sh.at[pl.ds(rbase + i * K, K)])

    def _ones(j, _):
        for k in range(D // 16):
            buf_v[j, pl.ds(16 * k, 16)] = jnp.ones((16,), jnp.float32)
        return 0

    lax.fori_loop(0, K, _ones, 0)
    plsc.subcore_barrier()

    def _ifetch(b, m):
        pltpu.async_copy(rc_hbm.at[gb0 + b], ir_v.at[m], si[m])

    def _iwait(m):
        pltpu.make_async_copy(rc_hbm.at[0], ir_v.at[m], si[m]).wait()

    def _swait(j):
        pltpu.make_async_copy(buf_v, acc_sh.at[ir_v.at[0, 1]], ss[j]).wait()

    _ifetch(0, 0)
    _ifetch(1, 1)

    def _body(i, _):
        for u in range(4):
            b = 4 * i + u

            if u < 2:
                @pl.when(i >= 1)
                def _():
                    _swait(u % 2)
            else:
                _swait(u % 2)

            @pl.when(b + 2 < NB)
            def _():
                _ifetch(b + 2, (u + 2) % 4)

            _iwait(u)
            pltpu.async_copy(buf_v, acc_sh.at[ir_v.at[u, 1]], ss[u % 2],
                             add=True)
        return 0

    lax.fori_loop(0, NB // 4, _body, 0)
    _swait(0)
    _swait(1)
    plsc.subcore_barrier()
    pltpu.sync_copy(acc_sh.at[pl.ds(rbase, RPT)],
                    degp_hbm.at[c].at[pl.ds(rbase, RPT)])


@functools.partial(
    pl.kernel,
    out_type=jax.ShapeDtypeStruct((NC, NPAD, D), jnp.float32),
    mesh=_sc_mesh,
    scratch_types=[
        pltpu.VMEM((8, 2, K), jnp.int32),
        pltpu.VMEM((K, D), jnp.float32),
        pltpu.VMEM((K, D), jnp.float32),
        pltpu.VMEM((K, D), jnp.float32),
        pltpu.VMEM((K, D), jnp.float32),
        pltpu.VMEM_SHARED((NPAD, D), jnp.float32),
        pltpu.SemaphoreType.DMA,
        pltpu.SemaphoreType.DMA,
        pltpu.SemaphoreType.DMA,
        pltpu.SemaphoreType.DMA,
        pltpu.SemaphoreType.DMA,
        pltpu.SemaphoreType.DMA,
        pltpu.SemaphoreType.DMA,
        pltpu.SemaphoreType.DMA,
        pltpu.SemaphoreType.DMA,
        pltpu.SemaphoreType.DMA,
        pltpu.SemaphoreType.DMA,
        pltpu.SemaphoreType.DMA,
        pltpu.SemaphoreType.DMA,
        pltpu.SemaphoreType.DMA,
        pltpu.SemaphoreType.DMA,
        pltpu.SemaphoreType.DMA,
    ],
)
def _prop_kernel(y_hbm, rc_hbm, p_hbm, ir_v, r0, r1, r2, r3, acc_sh,
                 si0, si1, si2, si3, si4, si5, si6, si7,
                 sg0, sg1, sg2, sg3, ss0, ss1, ss2, ss3):
    c = lax.axis_index("c")
    s = lax.axis_index("s")
    wid = s * NC + c
    gb0 = wid * NB
    bufs = (r0, r1, r2, r3)
    si = (si0, si1, si2, si3, si4, si5, si6, si7)
    sg = (sg0, sg1, sg2, sg3)
    ss = (ss0, ss1, ss2, ss3)

    def _zero(j, _):
        for k in range(D // 16):
            r0[j, pl.ds(16 * k, 16)] = jnp.zeros((16,), jnp.float32)
        return 0

    lax.fori_loop(0, K, _zero, 0)
    rbase = pl.multiple_of(s * RPT, 8)
    for i in range(RPT // K):
        pltpu.sync_copy(r0, acc_sh.at[pl.ds(rbase + i * K, K)])
    plsc.subcore_barrier()

    def _ifetch(b, m):
        pltpu.async_copy(rc_hbm.at[gb0 + b], ir_v.at[m], si[m])

    def _iwait(m):
        pltpu.make_async_copy(rc_hbm.at[0], ir_v.at[m], si[m]).wait()

    def _gather(m, j):
        pltpu.async_copy(y_hbm.at[ir_v.at[m, 0]], bufs[j], sg[j])

    def _gwait(j):
        pltpu.make_async_copy(y_hbm.at[ir_v.at[0, 0]], bufs[j], sg[j]).wait()

    def _scatter(m, j):
        pltpu.async_copy(bufs[j], acc_sh.at[ir_v.at[m, 1]], ss[j], add=True)

    def _swait(j):
        pltpu.make_async_copy(bufs[j], acc_sh.at[ir_v.at[0, 1]], ss[j]).wait()

    for m in range(6):
        _ifetch(m, m)
    _iwait(0)
    _gather(0, 0)
    _iwait(1)
    _gather(1, 1)

    def _body(i, _):
        for u in range(8):
            b = 8 * i + u
            jn = (u + 2) % 4
            mn = (u + 2) % 8

            if u < 2:
                @pl.when(i >= 1)
                def _():
                    _swait(jn)
            else:
                _swait(jn)

            @pl.when(b + 6 < NB)
            def _():
                _ifetch(b + 6, (u + 6) % 8)

            @pl.when(b + 2 < NB)
            def _():
                _iwait(mn)
                _gather(mn, jn)

            _gwait(u % 4)
            _scatter(u % 8, u % 4)
        return 0

    lax.fori_loop(0, NB // 8, _body, 0)
    _swait((NB - 2) % 4)
    _swait((NB - 1) % 4)
    plsc.subcore_barrier()
    pltpu.sync_copy(acc_sh.at[pl.ds(rbase, RPT)],
                    p_hbm.at[c].at[pl.ds(rbase, RPT)])


def _s16_body(dp_ref, s_ref):
    d = dp_ref[0][:, 0:16] + dp_ref[1][:, 0:16]
    s_ref[...] = jnp.where(d > 0, lax.rsqrt(jnp.where(d > 0, d, 1.0)), 0.0)


def _s16_call(degp):
    return pl.pallas_call(
        _s16_body,
        grid=(NPAD // 1024,),
        in_specs=[pl.BlockSpec((2, 1024, D), lambda i: (0, i, 0))],
        out_specs=pl.BlockSpec((1024, 16), lambda i: (i, 0)),
        out_shape=jax.ShapeDtypeStruct((NPAD, 16), jnp.float32),
    )(degp)


def _y_body(h_ref, s_ref, y_ref):
    y_ref[...] = h_ref[...] * s_ref[:, 0:1]


def _y_call(h, s16):
    return pl.pallas_call(
        _y_body,
        grid=(10,),
        in_specs=[
            pl.BlockSpec((1000, D), lambda i: (i, 0)),
            pl.BlockSpec((1000, 16), lambda i: (i, 0)),
        ],
        out_specs=pl.BlockSpec((1000, D), lambda i: (i, 0)),
        out_shape=jax.ShapeDtypeStruct((N, D), jnp.float32),
    )(h, s16)


def _combine_body(p_ref, s_ref, o_ref, onew_ref, y_ref):
    sc = s_ref[:, 0:1]
    x = sc * (p_ref[0] + p_ref[1])
    onew_ref[...] = o_ref[...] + x
    y_ref[...] = sc * x


def _combine_call(p, s16, out):
    return pl.pallas_call(
        _combine_body,
        grid=(10,),
        in_specs=[
            pl.BlockSpec((2, 1000, D), lambda i: (0, i, 0)),
            pl.BlockSpec((1000, 16), lambda i: (i, 0)),
            pl.BlockSpec((1000, D), lambda i: (i, 0)),
        ],
        out_specs=[
            pl.BlockSpec((1000, D), lambda i: (i, 0)),
            pl.BlockSpec((1000, D), lambda i: (i, 0)),
        ],
        out_shape=[
            jax.ShapeDtypeStruct((N, D), jnp.float32),
            jax.ShapeDtypeStruct((N, D), jnp.float32),
        ],
    )(p, s16, out)


def _final_body(p_ref, s_ref, o_ref, z_ref):
    sc = s_ref[:, 0:1]
    z_ref[...] = ALPHA * (o_ref[...] + sc * (p_ref[0] + p_ref[1]))


def _final_call(p, s16, out):
    return pl.pallas_call(
        _final_body,
        grid=(10,),
        in_specs=[
            pl.BlockSpec((2, 1000, D), lambda i: (0, i, 0)),
            pl.BlockSpec((1000, 16), lambda i: (i, 0)),
            pl.BlockSpec((1000, D), lambda i: (i, 0)),
        ],
        out_specs=pl.BlockSpec((1000, D), lambda i: (i, 0)),
        out_shape=jax.ShapeDtypeStruct((N, D), jnp.float32),
    )(p, s16, out)


@jax.jit
def _impl(x_user, x_item, W_user, b_user, W_item, b_item, eui, eiu):
    X = jnp.stack([x_user, x_item])
    W = jnp.stack([W_user, W_item])
    b = jnp.stack([b_user, b_item])[:, None, :]
    h = _lin_call(X, W, b)

    npad = EPAD - E
    row = jnp.concatenate(
        [eui[0], eiu[0] + N_USER, jnp.zeros((npad,), jnp.int32)]
    )
    col = jnp.concatenate(
        [eui[1] + N_USER, eiu[1], jnp.full((npad,), N, jnp.int32)]
    )
    rc = jnp.stack([row.reshape(NW * NB, K), col.reshape(NW * NB, K)], axis=1)

    degp = _deg_kernel(rc)
    s16 = _s16_call(degp)
    y = _y_call(h, s16)

    out = h
    for layer in range(NUM_LAYERS):
        p = _prop_kernel(y, rc)
        if layer < NUM_LAYERS - 1:
            out, y = _combine_call(p, s16, out)
        else:
            z = _final_call(p, s16, out)
    return z[:N_USER], z[N_USER:]


def kernel(x_user, x_item, W_user, b_user, W_item, b_item,
           edge_user_item, edge_item_user):
    return _impl(x_user, x_item, W_user, b_user, W_item, b_item,
                 edge_user_item, edge_item_user)


# speedup vs baseline: 18.1603x; 3.2245x over previous
import functools

import jax
import jax.numpy as jnp
from jax import lax
from jax.experimental import pallas as pl
from jax.experimental.pallas import tpu as pltpu
from jax.experimental.pallas import tpu_sc as plsc

N_USER = 5000
N_ITEM = 5000
D = 128
E_PER = 160000
NUM_LAYERS = 3
ALPHA = 1.0 / (NUM_LAYERS + 1)

NC = 2
NS = 16
NW = NC * NS
K = 32
EPT = 10240
NB = EPT // K
HALF = 5120
NPAD = 2 * HALF
RPH = HALF // NS

_sc_mesh = plsc.VectorSubcoreMesh(core_axis_name="c", subcore_axis_name="s")


def _lin_body(x_ref, w_ref, b_ref, o_ref):
    o_ref[...] = (
        jnp.dot(x_ref[0], w_ref[0], preferred_element_type=jnp.float32)
        + b_ref[0]
    )


def _lin_call(X, W, b):
    return pl.pallas_call(
        _lin_body,
        grid=(2, 5),
        in_specs=[
            pl.BlockSpec((1, 1024, D), lambda t, j: (t, j, 0)),
            pl.BlockSpec((1, D, D), lambda t, j: (t, 0, 0)),
            pl.BlockSpec((1, 1, D), lambda t, j: (t, 0, 0)),
        ],
        out_specs=pl.BlockSpec((1024, D), lambda t, j: (t * 5 + j, 0)),
        out_shape=jax.ShapeDtypeStruct((NPAD, D), jnp.float32),
    )(X, W, b)


@functools.partial(
    pl.kernel,
    out_type=jax.ShapeDtypeStruct((NC, HALF, D), jnp.float32),
    mesh=_sc_mesh,
    scratch_types=[
        pltpu.VMEM((4, 2, K), jnp.int32),
        pltpu.VMEM((K, D), jnp.float32),
        pltpu.VMEM_SHARED((HALF, D), jnp.float32),
        pltpu.SemaphoreType.DMA,
        pltpu.SemaphoreType.DMA,
        pltpu.SemaphoreType.DMA,
        pltpu.SemaphoreType.DMA,
        pltpu.SemaphoreType.DMA,
        pltpu.SemaphoreType.DMA,
    ],
)
def _deg_kernel(rc_hbm, degp_hbm, ir_v, buf_v, acc_sh,
                si0, si1, si2, si3, ss0, ss1):
    c = lax.axis_index("c")
    s = lax.axis_index("s")
    gb0 = (c * NS + s) * NB
    si = (si0, si1, si2, si3)
    ss = (ss0, ss1)

    def _zero(j, _):
        for k in range(D // 16):
            buf_v[j, pl.ds(16 * k, 16)] = jnp.zeros((16,), jnp.float32)
        return 0

    lax.fori_loop(0, K, _zero, 0)
    rbase = pl.multiple_of(s * RPH, 8)
    for i in range(RPH // K):
        pltpu.sync_copy(buf_v, acc_sh.at[pl.ds(rbase + i * K, K)])

    def _ones(j, _):
        for k in range(D // 16):
            buf_v[j, pl.ds(16 * k, 16)] = jnp.ones((16,), jnp.float32)
        return 0

    lax.fori_loop(0, K, _ones, 0)
    plsc.subcore_barrier()

    def _ifetch(b, m):
        pltpu.async_copy(rc_hbm.at[gb0 + b], ir_v.at[m], si[m])

    def _iwait(m):
        pltpu.make_async_copy(rc_hbm.at[0], ir_v.at[m], si[m]).wait()

    def _swait(j):
        pltpu.make_async_copy(buf_v, acc_sh.at[ir_v.at[0, 1]], ss[j]).wait()

    _ifetch(0, 0)
    _ifetch(1, 1)

    def _body(i, _):
        for u in range(4):
            b = 4 * i + u

            if u < 2:
                @pl.when(i >= 1)
                def _():
                    _swait(u % 2)
            else:
                _swait(u % 2)

            @pl.when(b + 2 < NB)
            def _():
                _ifetch(b + 2, (u + 2) % 4)

            _iwait(u)
            pltpu.async_copy(buf_v, acc_sh.at[ir_v.at[u, 1]], ss[u % 2],
                             add=True)
        return 0

    lax.fori_loop(0, NB // 4, _body, 0)
    _swait(0)
    _swait(1)
    plsc.subcore_barrier()
    pltpu.sync_copy(acc_sh.at[pl.ds(rbase, RPH)],
                    degp_hbm.at[c].at[pl.ds(rbase, RPH)])


@functools.partial(
    pl.kernel,
    out_type=jax.ShapeDtypeStruct((NC, HALF, D), jnp.float32),
    mesh=_sc_mesh,
    scratch_types=[
        pltpu.VMEM((8, 2, K), jnp.int32),
        pltpu.VMEM((K, D), jnp.float32),
        pltpu.VMEM((K, D), jnp.float32),
        pltpu.VMEM((K, D), jnp.float32),
        pltpu.VMEM((K, D), jnp.float32),
        pltpu.VMEM_SHARED((HALF, D), jnp.float32),
        pltpu.VMEM_SHARED((HALF, D), jnp.float32),
        pltpu.SemaphoreType.DMA,
        pltpu.SemaphoreType.DMA,
        pltpu.SemaphoreType.DMA,
        pltpu.SemaphoreType.DMA,
        pltpu.SemaphoreType.DMA,
        pltpu.SemaphoreType.DMA,
        pltpu.SemaphoreType.DMA,
        pltpu.SemaphoreType.DMA,
        pltpu.SemaphoreType.DMA,
        pltpu.SemaphoreType.DMA,
        pltpu.SemaphoreType.DMA,
        pltpu.SemaphoreType.DMA,
        pltpu.SemaphoreType.DMA,
        pltpu.SemaphoreType.DMA,
        pltpu.SemaphoreType.DMA,
        pltpu.SemaphoreType.DMA,
    ],
)
def _prop_kernel(y_hbm, rc_hbm, p_hbm, ir_v, r0, r1, r2, r3, ysh, acc_sh,
                 si0, si1, si2, si3, si4, si5, si6, si7,
                 sg0, sg1, sg2, sg3, ss0, ss1, ss2, ss3):
    c = lax.axis_index("c")
    s = lax.axis_index("s")
    gb0 = (c * NS + s) * NB
    bufs = (r0, r1, r2, r3)
    si = (si0, si1, si2, si3, si4, si5, si6, si7)
    sg = (sg0, sg1, sg2, sg3)
    ss = (ss0, ss1, ss2, ss3)

    def _zero(j, _):
        for k in range(D // 16):
            r0[j, pl.ds(16 * k, 16)] = jnp.zeros((16,), jnp.float32)
        return 0

    lax.fori_loop(0, K, _zero, 0)
    rbase = pl.multiple_of(s * RPH, 8)
    for i in range(RPH // K):
        pltpu.sync_copy(r0, acc_sh.at[pl.ds(rbase + i * K, K)])
    ybase = pl.multiple_of(c * HALF + s * RPH, 8)
    pltpu.sync_copy(y_hbm.at[pl.ds(ybase, RPH)], ysh.at[pl.ds(rbase, RPH)])
    plsc.subcore_barrier()

    def _ifetch(b, m):
        pltpu.async_copy(rc_hbm.at[gb0 + b], ir_v.at[m], si[m])

    def _iwait(m):
        pltpu.make_async_copy(rc_hbm.at[0], ir_v.at[m], si[m]).wait()

    def _gather(m, j):
        pltpu.async_copy(ysh.at[ir_v.at[m, 0]], bufs[j], sg[j])

    def _gwait(j):
        pltpu.make_async_copy(ysh.at[ir_v.at[0, 0]], bufs[j], sg[j]).wait()

    def _scatter(m, j):
        pltpu.async_copy(bufs[j], acc_sh.at[ir_v.at[m, 1]], ss[j], add=True)

    def _swait(j):
        pltpu.make_async_copy(bufs[j], acc_sh.at[ir_v.at[0, 1]], ss[j]).wait()

    for m in range(6):
        _ifetch(m, m)
    _iwait(0)
    _gather(0, 0)
    _iwait(1)
    _gather(1, 1)

    def _body(i, _):
        for u in range(8):
            b = 8 * i + u
            jn = (u + 2) % 4
            mn = (u + 2) % 8

            if u < 2:
                @pl.when(i >= 1)
                def _():
                    _swait(jn)
            else:
                _swait(jn)

            @pl.when(b + 6 < NB)
            def _():
                _ifetch(b + 6, (u + 6) % 8)

            @pl.when(b + 2 < NB)
            def _():
                _iwait(mn)
                _gather(mn, jn)

            _gwait(u % 4)
            _scatter(u % 8, u % 4)
        return 0

    lax.fori_loop(0, NB // 8, _body, 0)
    _swait((NB - 2) % 4)
    _swait((NB - 1) % 4)
    plsc.subcore_barrier()
    pltpu.sync_copy(acc_sh.at[pl.ds(rbase, RPH)],
                    p_hbm.at[c].at[pl.ds(rbase, RPH)])


def _side_map(i):
    return (1 - i // 5, i % 5, 0)


def _s16_body(dp_ref, s_ref):
    d = dp_ref[0][:, 0:16]
    s_ref[...] = jnp.where(d > 0, lax.rsqrt(jnp.where(d > 0, d, 1.0)), 0.0)


def _s16_call(degp):
    return pl.pallas_call(
        _s16_body,
        grid=(10,),
        in_specs=[pl.BlockSpec((1, 1024, D), _side_map)],
        out_specs=pl.BlockSpec((1024, 16), lambda i: (i, 0)),
        out_shape=jax.ShapeDtypeStruct((NPAD, 16), jnp.float32),
    )(degp)


def _y_body(h_ref, s_ref, y_ref):
    y_ref[...] = h_ref[...] * s_ref[:, 0:1]


def _y_call(h, s16):
    return pl.pallas_call(
        _y_body,
        grid=(10,),
        in_specs=[
            pl.BlockSpec((1024, D), lambda i: (i, 0)),
            pl.BlockSpec((1024, 16), lambda i: (i, 0)),
        ],
        out_specs=pl.BlockSpec((1024, D), lambda i: (i, 0)),
        out_shape=jax.ShapeDtypeStruct((NPAD, D), jnp.float32),
    )(h, s16)


def _combine_body(p_ref, s_ref, o_ref, onew_ref, y_ref):
    sc = s_ref[:, 0:1]
    x = sc * p_ref[0]
    onew_ref[...] = o_ref[...] + x
    y_ref[...] = sc * x


def _combine_call(p, s16, out):
    return pl.pallas_call(
        _combine_body,
        grid=(10,),
        in_specs=[
            pl.BlockSpec((1, 1024, D), _side_map),
            pl.BlockSpec((1024, 16), lambda i: (i, 0)),
            pl.BlockSpec((1024, D), lambda i: (i, 0)),
        ],
        out_specs=[
            pl.BlockSpec((1024, D), lambda i: (i, 0)),
            pl.BlockSpec((1024, D), lambda i: (i, 0)),
        ],
        out_shape=[
            jax.ShapeDtypeStruct((NPAD, D), jnp.float32),
            jax.ShapeDtypeStruct((NPAD, D), jnp.float32),
        ],
    )(p, s16, out)


def _final_body(p_ref, s_ref, o_ref, z_ref):
    sc = s_ref[:, 0:1]
    z_ref[...] = ALPHA * (o_ref[...] + sc * p_ref[0])


def _final_call(p, s16, out):
    return pl.pallas_call(
        _final_body,
        grid=(10,),
        in_specs=[
            pl.BlockSpec((1, 1024, D), _side_map),
            pl.BlockSpec((1024, 16), lambda i: (i, 0)),
            pl.BlockSpec((1024, D), lambda i: (i, 0)),
        ],
        out_specs=pl.BlockSpec((1024, D), lambda i: (i, 0)),
        out_shape=jax.ShapeDtypeStruct((NPAD, D), jnp.float32),
    )(p, s16, out)


@jax.jit
def _impl(x_user, x_item, W_user, b_user, W_item, b_item, eui, eiu):
    pad = ((0, HALF - N_USER), (0, 0))
    X = jnp.stack([jnp.pad(x_user, pad), jnp.pad(x_item, pad)])
    W = jnp.stack([W_user, W_item])
    b = jnp.stack([b_user, b_item])[:, None, :]
    h = _lin_call(X, W, b)

    epad = NS * EPT - E_PER
    zpad = jnp.zeros((epad,), jnp.int32)
    dpad = jnp.full((epad,), N_USER, jnp.int32)
    row = jnp.concatenate([eui[0], zpad, eiu[0], zpad])
    col = jnp.concatenate([eui[1], dpad, eiu[1], dpad])
    rc = jnp.stack([row.reshape(NW * NB, K), col.reshape(NW * NB, K)], axis=1)

    degp = _deg_kernel(rc)
    s16 = _s16_call(degp)
    y = _y_call(h, s16)

    out = h
    for layer in range(NUM_LAYERS):
        p = _prop_kernel(y, rc)
        if layer < NUM_LAYERS - 1:
            out, y = _combine_call(p, s16, out)
        else:
            z = _final_call(p, s16, out)
    return z[:N_USER], z[HALF:HALF + N_ITEM]


def kernel(x_user, x_item, W_user, b_user, W_item, b_item,
           edge_user_item, edge_item_user):
    return _impl(x_user, x_item, W_user, b_user, W_item, b_item,
                 edge_user_item, edge_item_user)


# speedup vs baseline: 18.4599x; 1.0165x over previous
import functools

import jax
import jax.numpy as jnp
from jax import lax
from jax.experimental import pallas as pl
from jax.experimental.pallas import tpu as pltpu
from jax.experimental.pallas import tpu_sc as plsc

N_USER = 5000
N_ITEM = 5000
D = 128
E_PER = 160000
NUM_LAYERS = 3
ALPHA = 1.0 / (NUM_LAYERS + 1)

NC = 2
NS = 16
NW = NC * NS
K = 32
EPT = 10240
NB = EPT // K
HALF = 5120
NPAD = 2 * HALF
RPH = HALF // NS

_sc_mesh = plsc.VectorSubcoreMesh(core_axis_name="c", subcore_axis_name="s")


def _lin_body(x_ref, w_ref, b_ref, o_ref):
    o_ref[...] = (
        jnp.dot(x_ref[0], w_ref[0], preferred_element_type=jnp.float32)
        + b_ref[0]
    )


def _lin_call(X, W, b):
    return pl.pallas_call(
        _lin_body,
        grid=(2, 5),
        in_specs=[
            pl.BlockSpec((1, 1024, D), lambda t, j: (t, j, 0)),
            pl.BlockSpec((1, D, D), lambda t, j: (t, 0, 0)),
            pl.BlockSpec((1, 1, D), lambda t, j: (t, 0, 0)),
        ],
        out_specs=pl.BlockSpec((1024, D), lambda t, j: (t * 5 + j, 0)),
        out_shape=jax.ShapeDtypeStruct((NPAD, D), jnp.float32),
    )(X, W, b)


@functools.partial(
    pl.kernel,
    out_type=jax.ShapeDtypeStruct((NC, HALF, D), jnp.float32),
    mesh=_sc_mesh,
    scratch_types=[
        pltpu.VMEM((4, 2, K), jnp.int32),
        pltpu.VMEM((K, D), jnp.float32),
        pltpu.VMEM_SHARED((HALF, D), jnp.float32),
        pltpu.SemaphoreType.DMA,
        pltpu.SemaphoreType.DMA,
        pltpu.SemaphoreType.DMA,
        pltpu.SemaphoreType.DMA,
        pltpu.SemaphoreType.DMA,
        pltpu.SemaphoreType.DMA,
    ],
)
def _deg_kernel(rc_hbm, degp_hbm, ir_v, buf_v, acc_sh,
                si0, si1, si2, si3, ss0, ss1):
    c = lax.axis_index("c")
    s = lax.axis_index("s")
    gb0 = (c * NS + s) * NB
    si = (si0, si1, si2, si3)
    ss = (ss0, ss1)

    def _zero(j, _):
        for k in range(D // 16):
            buf_v[j, pl.ds(16 * k, 16)] = jnp.zeros((16,), jnp.float32)
        return 0

    lax.fori_loop(0, K, _zero, 0)
    rbase = pl.multiple_of(s * RPH, 8)
    for i in range(RPH // K):
        pltpu.sync_copy(buf_v, acc_sh.at[pl.ds(rbase + i * K, K)])

    def _ones(j, _):
        for k in range(D // 16):
            buf_v[j, pl.ds(16 * k, 16)] = jnp.ones((16,), jnp.float32)
        return 0

    lax.fori_loop(0, K, _ones, 0)
    plsc.subcore_barrier()

    def _ifetch(b, m):
        pltpu.async_copy(rc_hbm.at[gb0 + b], ir_v.at[m], si[m])

    def _iwait(m):
        pltpu.make_async_copy(rc_hbm.at[0], ir_v.at[m], si[m]).wait()

    def _swait(j):
        pltpu.make_async_copy(buf_v, acc_sh.at[ir_v.at[0, 1]], ss[j]).wait()

    _ifetch(0, 0)
    _ifetch(1, 1)

    def _body(i, _):
        for u in range(4):
            b = 4 * i + u

            if u < 2:
                @pl.when(i >= 1)
                def _():
                    _swait(u % 2)
            else:
                _swait(u % 2)

            @pl.when(b + 2 < NB)
            def _():
                _ifetch(b + 2, (u + 2) % 4)

            _iwait(u)
            pltpu.async_copy(buf_v, acc_sh.at[ir_v.at[u, 1]], ss[u % 2],
                             add=True)
        return 0

    lax.fori_loop(0, NB // 4, _body, 0)
    _swait(0)
    _swait(1)
    plsc.subcore_barrier()
    pltpu.sync_copy(acc_sh.at[pl.ds(rbase, RPH)],
                    degp_hbm.at[c].at[pl.ds(rbase, RPH)])


@functools.partial(
    pl.kernel,
    out_type=jax.ShapeDtypeStruct((NC, HALF, D), jnp.float32),
    mesh=_sc_mesh,
    scratch_types=[
        pltpu.VMEM((8, 2, K), jnp.int32),
        pltpu.VMEM((K, D), jnp.float32),
        pltpu.VMEM((K, D), jnp.float32),
        pltpu.VMEM((K, D), jnp.float32),
        pltpu.VMEM((K, D), jnp.float32),
        pltpu.VMEM_SHARED((HALF, D), jnp.float32),
        pltpu.VMEM_SHARED((HALF, D), jnp.float32),
        pltpu.SemaphoreType.DMA,
        pltpu.SemaphoreType.DMA,
        pltpu.SemaphoreType.DMA,
        pltpu.SemaphoreType.DMA,
        pltpu.SemaphoreType.DMA,
        pltpu.SemaphoreType.DMA,
        pltpu.SemaphoreType.DMA,
        pltpu.SemaphoreType.DMA,
        pltpu.SemaphoreType.DMA,
        pltpu.SemaphoreType.DMA,
        pltpu.SemaphoreType.DMA,
        pltpu.SemaphoreType.DMA,
        pltpu.SemaphoreType.DMA,
        pltpu.SemaphoreType.DMA,
        pltpu.SemaphoreType.DMA,
        pltpu.SemaphoreType.DMA,
    ],
)
def _prop_kernel(y_hbm, rc_hbm, p_hbm, ir_v, r0, r1, r2, r3, ysh, acc_sh,
                 si0, si1, si2, si3, si4, si5, si6, si7,
                 sg0, sg1, sg2, sg3, ss0, ss1, ss2, ss3):
    c = lax.axis_index("c")
    s = lax.axis_index("s")
    gb0 = (c * NS + s) * NB
    bufs = (r0, r1, r2, r3)
    si = (si0, si1, si2, si3, si4, si5, si6, si7)
    sg = (sg0, sg1, sg2, sg3)
    ss = (ss0, ss1, ss2, ss3)

    def _zero(j, _):
        for k in range(D // 16):
            r0[j, pl.ds(16 * k, 16)] = jnp.zeros((16,), jnp.float32)
        return 0

    lax.fori_loop(0, K, _zero, 0)
    rbase = pl.multiple_of(s * RPH, 8)
    for i in range(RPH // K):
        pltpu.sync_copy(r0, acc_sh.at[pl.ds(rbase + i * K, K)])
    ybase = pl.multiple_of(c * HALF + s * RPH, 8)
    pltpu.sync_copy(y_hbm.at[pl.ds(ybase, RPH)], ysh.at[pl.ds(rbase, RPH)])
    plsc.subcore_barrier()

    def _ifetch(b, m):
        pltpu.async_copy(rc_hbm.at[gb0 + b], ir_v.at[m], si[m])

    def _iwait(m):
        pltpu.make_async_copy(rc_hbm.at[0], ir_v.at[m], si[m]).wait()

    def _gather(m, j):
        pltpu.async_copy(ysh.at[ir_v.at[m, 0]], bufs[j], sg[j])

    def _gwait(j):
        pltpu.make_async_copy(ysh.at[ir_v.at[0, 0]], bufs[j], sg[j]).wait()

    def _scatter(m, j):
        pltpu.async_copy(bufs[j], acc_sh.at[ir_v.at[m, 1]], ss[j], add=True)

    def _swait(j):
        pltpu.make_async_copy(bufs[j], acc_sh.at[ir_v.at[0, 1]], ss[j]).wait()

    for m in range(6):
        _ifetch(m, m)
    _iwait(0)
    _gather(0, 0)
    _iwait(1)
    _gather(1, 1)

    def _body(i, _):
        for u in range(8):
            b = 8 * i + u
            jn = (u + 2) % 4
            mn = (u + 2) % 8

            if u < 2:
                @pl.when(i >= 1)
                def _():
                    _swait(jn)
            else:
                _swait(jn)

            @pl.when(b + 6 < NB)
            def _():
                _ifetch(b + 6, (u + 6) % 8)

            @pl.when(b + 2 < NB)
            def _():
                _iwait(mn)
                _gather(mn, jn)

            _gwait(u % 4)
            _scatter(u % 8, u % 4)
        return 0

    lax.fori_loop(0, NB // 8, _body, 0)
    _swait((NB - 2) % 4)
    _swait((NB - 1) % 4)
    plsc.subcore_barrier()
    pltpu.sync_copy(acc_sh.at[pl.ds(rbase, RPH)],
                    p_hbm.at[c].at[pl.ds(rbase, RPH)])


def _side_map(i):
    return (1 - i // 5, i % 5, 0)


def _sy_body(dp_ref, h_ref, s_ref, y_ref):
    d = dp_ref[0][:, 0:16]
    s = jnp.where(d > 0, lax.rsqrt(jnp.where(d > 0, d, 1.0)), 0.0)
    s_ref[...] = s
    y_ref[...] = h_ref[...] * s[:, 0:1]


def _sy_call(degp, h):
    return pl.pallas_call(
        _sy_body,
        grid=(10,),
        in_specs=[
            pl.BlockSpec((1, 1024, D), _side_map),
            pl.BlockSpec((1024, D), lambda i: (i, 0)),
        ],
        out_specs=[
            pl.BlockSpec((1024, 16), lambda i: (i, 0)),
            pl.BlockSpec((1024, D), lambda i: (i, 0)),
        ],
        out_shape=[
            jax.ShapeDtypeStruct((NPAD, 16), jnp.float32),
            jax.ShapeDtypeStruct((NPAD, D), jnp.float32),
        ],
    )(degp, h)


def _combine_body(p_ref, s_ref, o_ref, onew_ref, y_ref):
    sc = s_ref[:, 0:1]
    x = sc * p_ref[0]
    onew_ref[...] = o_ref[...] + x
    y_ref[...] = sc * x


def _combine_call(p, s16, out):
    return pl.pallas_call(
        _combine_body,
        grid=(10,),
        in_specs=[
            pl.BlockSpec((1, 1024, D), _side_map),
            pl.BlockSpec((1024, 16), lambda i: (i, 0)),
            pl.BlockSpec((1024, D), lambda i: (i, 0)),
        ],
        out_specs=[
            pl.BlockSpec((1024, D), lambda i: (i, 0)),
            pl.BlockSpec((1024, D), lambda i: (i, 0)),
        ],
        out_shape=[
            jax.ShapeDtypeStruct((NPAD, D), jnp.float32),
            jax.ShapeDtypeStruct((NPAD, D), jnp.float32),
        ],
    )(p, s16, out)


def _final_body(p_ref, s_ref, o_ref, z_ref):
    sc = s_ref[:, 0:1]
    z_ref[...] = ALPHA * (o_ref[...] + sc * p_ref[0])


def _final_call(p, s16, out):
    return pl.pallas_call(
        _final_body,
        grid=(10,),
        in_specs=[
            pl.BlockSpec((1, 1024, D), _side_map),
            pl.BlockSpec((1024, 16), lambda i: (i, 0)),
            pl.BlockSpec((1024, D), lambda i: (i, 0)),
        ],
        out_specs=pl.BlockSpec((1024, D), lambda i: (i, 0)),
        out_shape=jax.ShapeDtypeStruct((NPAD, D), jnp.float32),
    )(p, s16, out)


@jax.jit
def _impl(x_user, x_item, W_user, b_user, W_item, b_item, eui, eiu):
    pad = ((0, HALF - N_USER), (0, 0))
    X = jnp.stack([jnp.pad(x_user, pad), jnp.pad(x_item, pad)])
    W = jnp.stack([W_user, W_item])
    b = jnp.stack([b_user, b_item])[:, None, :]
    h = _lin_call(X, W, b)

    epad = NS * EPT - E_PER
    zpad = jnp.zeros((epad,), jnp.int32)
    dpad = jnp.full((epad,), N_USER, jnp.int32)
    row = jnp.concatenate([eui[0], zpad, eiu[0], zpad])
    col = jnp.concatenate([eui[1], dpad, eiu[1], dpad])
    rc = jnp.stack([row.reshape(NW * NB, K), col.reshape(NW * NB, K)], axis=1)

    degp = _deg_kernel(rc)
    s16, y = _sy_call(degp, h)

    out = h
    for layer in range(NUM_LAYERS):
        p = _prop_kernel(y, rc)
        if layer < NUM_LAYERS - 1:
            out, y = _combine_call(p, s16, out)
        else:
            z = _final_call(p, s16, out)
    return z[:N_USER], z[HALF:HALF + N_ITEM]


def kernel(x_user, x_item, W_user, b_user, W_item, b_item,
           edge_user_item, edge_item_user):
    return _impl(x_user, x_item, W_user, b_user, W_item, b_item,
                 edge_user_item, edge_item_user)
